# Initial kernel scaffold; baseline (speedup 1.0000x reference)
import functools

import jax
import jax.numpy as jnp
from jax import lax
from jax.experimental import pallas as pl
from jax.experimental.pallas import tpu as pltpu
from jax.experimental.pallas import tpu_sc as plsc

N_NODES = 10000
N_EDGES = 160000
NS = 16
CHUNK = 80
EPT = N_EDGES // NS
NCH = EPT // CHUNK
ROWS_PT = N_NODES // NS
ZROWS = 125
F32 = jnp.float32


def _fill_zeros_2d(buf, rows, cols):
  z16 = jnp.zeros((16,), F32)

  @pl.loop(0, rows)
  def _(r):
    @pl.loop(0, cols // 16)
    def _(k):
      buf[r, pl.ds(k * 16, 16)] = z16


def _fill_1d(buf, n, value):
  v16 = jnp.full((16,), value, F32)

  @pl.loop(0, n // 16)
  def _(k):
    buf[pl.ds(k * 16, 16)] = v16


def _make_sc_scatter(d_half, with_deg):
  mesh = plsc.VectorSubcoreMesh(core_axis_name="c", subcore_axis_name="s")
  out_type = [
      jax.ShapeDtypeStruct((N_NODES, d_half), F32),
      jax.ShapeDtypeStruct((N_NODES, d_half), F32),
  ]
  scratch = [
      pltpu.VMEM((NCH, CHUNK), jnp.int32),
      pltpu.VMEM((NCH, CHUNK), jnp.int32),
      pltpu.VMEM((CHUNK, d_half), F32),
      pltpu.VMEM((ZROWS, d_half), F32),
      pltpu.VMEM_SHARED((N_NODES, d_half), F32),
      pltpu.SemaphoreType.DMA,
  ]
  if with_deg:
    out_type.append(jax.ShapeDtypeStruct((N_NODES,), F32))
    scratch += [
        pltpu.VMEM((CHUNK,), F32),
        pltpu.VMEM((2000,), F32),
        pltpu.VMEM_SHARED((N_NODES,), F32),
    ]

  def body(tab0, tab1, src_hbm, dst_hbm, out0, out1, *rest):
    if with_deg:
      deg_out = rest[0]
      src_v, dst_v, rows_v, zbuf, acc, sem, ones_v, dz_v, dacc = rest[1:]
    else:
      src_v, dst_v, rows_v, zbuf, acc, sem = rest

    c = lax.axis_index("c")
    s = lax.axis_index("s")

    _fill_zeros_2d(zbuf, ZROWS, d_half)

    @pl.loop(0, ROWS_PT // ZROWS)
    def _(i):
      pltpu.sync_copy(zbuf, acc.at[pl.ds(s * ROWS_PT + i * ZROWS, ZROWS)])

    if with_deg:
      @pl.when(jnp.logical_and(c == 0, s == 0))
      def _():
        _fill_1d(dz_v, 2000, 0.0)

        @pl.loop(0, N_NODES // 2000)
        def _(i):
          pltpu.sync_copy(dz_v, dacc.at[pl.ds(i * 2000, 2000)])

      @pl.when(c == 0)
      def _():
        _fill_1d(ones_v, CHUNK, 1.0)

    pltpu.sync_copy(src_hbm.at[s], src_v)
    pltpu.sync_copy(dst_hbm.at[s], dst_v)

    plsc.subcore_barrier()

    def edge_loop(tab):
      @pl.loop(0, NCH)
      def _(j):
        pltpu.async_copy(tab.at[src_v.at[j]], rows_v, sem).wait()
        pltpu.sync_copy(rows_v, acc.at[dst_v.at[j]], add=True)

    @pl.when(c == 0)
    def _():
      edge_loop(tab0)

    @pl.when(c == 1)
    def _():
      edge_loop(tab1)

    if with_deg:
      @pl.when(c == 0)
      def _():
        @pl.loop(0, NCH)
        def _(j):
          pltpu.sync_copy(ones_v, dacc.at[dst_v.at[j]], add=True)

    plsc.subcore_barrier()

    r0 = s * ROWS_PT

    @pl.when(c == 0)
    def _():
      pltpu.sync_copy(acc.at[pl.ds(r0, ROWS_PT)], out0.at[pl.ds(r0, ROWS_PT)])

    @pl.when(c == 1)
    def _():
      pltpu.sync_copy(acc.at[pl.ds(r0, ROWS_PT)], out1.at[pl.ds(r0, ROWS_PT)])

    if with_deg:
      @pl.when(jnp.logical_and(c == 0, s == 0))
      def _():
        pltpu.sync_copy(dacc, deg_out)

  return pl.kernel(body, out_type=tuple(out_type), mesh=mesh,
                   scratch_types=tuple(scratch))


_sc_sage = _make_sc_scatter(128, with_deg=True)
_sc_cheb1 = _make_sc_scatter(32, with_deg=False)
_sc_cheb2 = _make_sc_scatter(16, with_deg=False)

_RB = 1000
_GRID = N_NODES // _RB
_HI = lax.Precision.HIGHEST


def _tc1_body(x_ref, w0_ref, w1_ref, y0_ref, y1_ref):
  x = x_ref[...]
  y0_ref[...] = jnp.dot(x, w0_ref[...], precision=_HI)
  y1_ref[...] = jnp.dot(x, w1_ref[...], precision=_HI)


def _tc2_body(a0_ref, a1_ref, y0_ref, y1_ref, deg_ref, b_ref, wc_ref,
              u0_ref, u2_ref, un0_ref, un1_ref, norm_ref):
  a = jnp.concatenate([a0_ref[...] + y0_ref[...],
                       a1_ref[...] + y1_ref[...]], axis=1)
  degb = deg_ref[...]
  h = a / (degb + 1.0) + b_ref[...]
  h = jnp.where(h > 0, h, 0.01 * h)
  p = jnp.dot(h, wc_ref[...], precision=_HI)
  norm = lax.rsqrt(jnp.maximum(degb, 1.0))
  u0_ref[...] = p[:, 0:32]
  u2_ref[...] = p[:, 64:96]
  un0_ref[...] = p[:, 32:64] * norm
  un1_ref[...] = p[:, 64:96] * norm
  norm_ref[...] = norm


def _tc3_body(s2a_ref, norm_ref, v0_ref, v1_ref):
  norm = norm_ref[...]
  v = -s2a_ref[...] * (norm * norm)
  v0_ref[...] = v[:, 0:16]
  v1_ref[...] = v[:, 16:32]


def _tc4_body(u0_ref, u2_ref, s1_ref, sb0_ref, sb1_ref, norm_ref,
              bc_ref, wf_ref, bf_ref, out_ref):
  norm = norm_ref[...]
  s2b = jnp.concatenate([sb0_ref[...], sb1_ref[...]], axis=1)
  h2 = (u0_ref[...] - (s1_ref[...] + 2.0 * s2b) * norm - u2_ref[...]
        + bc_ref[...])
  h2 = jnp.where(h2 > 0, h2, 0.01 * h2)
  out_ref[...] = jnp.dot(h2, wf_ref[...], precision=_HI) + bf_ref[...]


def _row_spec(cols):
  return pl.BlockSpec((_RB, cols), lambda i: (i, 0))


def _full_spec(shape):
  return pl.BlockSpec(shape, lambda i: tuple(0 for _ in shape))


_tc1 = pl.pallas_call(
    _tc1_body,
    grid=(_GRID,),
    in_specs=[_row_spec(256), _full_spec((256, 128)), _full_spec((256, 128))],
    out_specs=[_row_spec(128), _row_spec(128)],
    out_shape=[jax.ShapeDtypeStruct((N_NODES, 128), F32)] * 2,
)

_tc2 = pl.pallas_call(
    _tc2_body,
    grid=(_GRID,),
    in_specs=[_row_spec(128), _row_spec(128), _row_spec(128), _row_spec(128),
              _row_spec(1), _full_spec((1, 256)), _full_spec((256, 96))],
    out_specs=[_row_spec(32), _row_spec(32), _row_spec(32), _row_spec(32),
               _row_spec(1)],
    out_shape=[jax.ShapeDtypeStruct((N_NODES, 32), F32)] * 4
    + [jax.ShapeDtypeStruct((N_NODES, 1), F32)],
)

_tc3 = pl.pallas_call(
    _tc3_body,
    grid=(_GRID,),
    in_specs=[_row_spec(32), _row_spec(1)],
    out_specs=[_row_spec(16), _row_spec(16)],
    out_shape=[jax.ShapeDtypeStruct((N_NODES, 16), F32)] * 2,
)

_tc4 = pl.pallas_call(
    _tc4_body,
    grid=(_GRID,),
    in_specs=[_row_spec(32), _row_spec(32), _row_spec(32), _row_spec(16),
              _row_spec(16), _row_spec(1), _full_spec((1, 32)),
              _full_spec((32, 40)), _full_spec((1, 40))],
    out_specs=_row_spec(40),
    out_shape=jax.ShapeDtypeStruct((N_NODES, 40), F32),
)


def kernel(features, edge_index, W_sage, b_sage, W_cheb, b_cheb, W_fc, b_fc):
  ei = edge_index.astype(jnp.int32)
  src = ei[0].reshape(NS, NCH, CHUNK)
  dst = ei[1].reshape(NS, NCH, CHUNK)

  y0, y1 = _tc1(features, W_sage[:, :128], W_sage[:, 128:])
  agg0, agg1, deg = _sc_sage(y0, y1, src, dst)

  wc96 = jnp.concatenate([W_cheb[0:256], W_cheb[256:512], W_cheb[512:768]],
                         axis=1)
  u0, u2, un1, un2, norm = _tc2(agg0, agg1, y0, y1, deg.reshape(N_NODES, 1),
                                b_sage.reshape(1, 256), wc96)
  s1, s2a = _sc_cheb1(un1, un2, src, dst)
  v0, v1 = _tc3(s2a, norm)
  s2b0, s2b1 = _sc_cheb2(v0, v1, src, dst)
  out = _tc4(u0, u2, s1, s2b0, s2b1, norm, b_cheb.reshape(1, 32), W_fc,
             b_fc.reshape(1, 40))
  return out


# speedup vs baseline: 7.1461x; 7.1461x over previous
import functools

import jax
import jax.numpy as jnp
from jax import lax
from jax.experimental import pallas as pl
from jax.experimental.pallas import tpu as pltpu
from jax.experimental.pallas import tpu_sc as plsc

N_NODES = 10000
N_EDGES = 160000
NS = 16
CHUNK = 80
EPT = N_EDGES // NS
NCH = EPT // CHUNK
F32 = jnp.float32


def _fill_zeros_2d(buf, rows, cols):
  z16 = jnp.zeros((16,), F32)

  @pl.loop(0, rows)
  def _(r):
    @pl.loop(0, cols // 16)
    def _(k):
      buf[r, pl.ds(k * 16, 16)] = z16


def _fill_1d(buf, n, value):
  v16 = jnp.full((16,), value, F32)

  @pl.loop(0, n // 16)
  def _(k):
    buf[pl.ds(k * 16, 16)] = v16


def _make_sc_scatter(d_half, with_deg):
  mesh = plsc.VectorSubcoreMesh(core_axis_name="c", subcore_axis_name="s")
  out_type = [
      jax.ShapeDtypeStruct((N_NODES, d_half), F32),
      jax.ShapeDtypeStruct((N_NODES, d_half), F32),
  ]
  scratch = [
      pltpu.VMEM((NCH, CHUNK), jnp.int32),
      pltpu.VMEM((NCH, CHUNK), jnp.int32),
      pltpu.VMEM((CHUNK, d_half), F32),
      pltpu.VMEM((8, d_half), F32),
      pltpu.VMEM_SHARED((N_NODES, d_half), F32),
      pltpu.SemaphoreType.DMA,
  ]
  if with_deg:
    out_type.append(jax.ShapeDtypeStruct((N_NODES,), F32))
    scratch += [
        pltpu.VMEM((CHUNK,), F32),
        pltpu.VMEM((2000,), F32),
        pltpu.VMEM_SHARED((N_NODES,), F32),
    ]

  def body(tab0, tab1, src_hbm, dst_hbm, out0, out1, *rest):
    if with_deg:
      deg_out = rest[0]
      src_v, dst_v, rows_v, zbuf, acc, sem, ones_v, dz_v, dacc = rest[1:]
    else:
      src_v, dst_v, rows_v, zbuf, acc, sem = rest

    c = lax.axis_index("c")
    s = lax.axis_index("s")

    start = jnp.where(s < 2, s * 632, 1264 + (s - 2) * 624)
    ngroups = jnp.where(s < 2, 79, 78)

    _fill_zeros_2d(zbuf, 8, d_half)

    @pl.loop(0, ngroups)
    def _(i):
      pltpu.sync_copy(zbuf, acc.at[pl.ds(start + i * 8, 8)])

    if with_deg:
      @pl.when(jnp.logical_and(c == 0, s == 0))
      def _():
        _fill_1d(dz_v, 2000, 0.0)

        @pl.loop(0, N_NODES // 2000)
        def _(i):
          pltpu.sync_copy(dz_v, dacc.at[pl.ds(i * 2000, 2000)])

      @pl.when(c == 0)
      def _():
        _fill_1d(ones_v, CHUNK, 1.0)

    pltpu.sync_copy(src_hbm.at[s], src_v)
    pltpu.sync_copy(dst_hbm.at[s], dst_v)

    plsc.subcore_barrier()

    def edge_loop(tab):
      @pl.loop(0, NCH)
      def _(j):
        pltpu.async_copy(tab.at[src_v.at[j]], rows_v, sem).wait()
        pltpu.sync_copy(rows_v, acc.at[dst_v.at[j]], add=True)

    @pl.when(c == 0)
    def _():
      edge_loop(tab0)

    @pl.when(c == 1)
    def _():
      edge_loop(tab1)

    if with_deg:
      @pl.when(c == 0)
      def _():
        @pl.loop(0, NCH)
        def _(j):
          pltpu.sync_copy(ones_v, dacc.at[dst_v.at[j]], add=True)

    plsc.subcore_barrier()

    def writeout(out):
      @pl.when(s < 2)
      def _():
        pltpu.sync_copy(acc.at[pl.ds(start, 632)], out.at[pl.ds(start, 632)])

      @pl.when(s >= 2)
      def _():
        pltpu.sync_copy(acc.at[pl.ds(start, 624)], out.at[pl.ds(start, 624)])

    @pl.when(c == 0)
    def _():
      writeout(out0)

    @pl.when(c == 1)
    def _():
      writeout(out1)

    if with_deg:
      @pl.when(jnp.logical_and(c == 0, s == 0))
      def _():
        pltpu.sync_copy(dacc, deg_out)

  return pl.kernel(body, out_type=tuple(out_type), mesh=mesh,
                   scratch_types=tuple(scratch),
                   compiler_params=pltpu.CompilerParams(
                       use_tc_tiling_on_sc=False))


_sc_sage = _make_sc_scatter(128, with_deg=True)
_sc_cheb1 = _make_sc_scatter(32, with_deg=False)
_sc_cheb2 = _make_sc_scatter(16, with_deg=False)

_RB = 1000
_GRID = N_NODES // _RB
_HI = lax.Precision.HIGHEST


def _tc1_body(x_ref, w0_ref, w1_ref, y0_ref, y1_ref):
  x = x_ref[...]
  y0_ref[...] = jnp.dot(x, w0_ref[...], precision=_HI)
  y1_ref[...] = jnp.dot(x, w1_ref[...], precision=_HI)


def _tc2_body(a0_ref, a1_ref, y0_ref, y1_ref, deg_ref, b_ref, wc_ref,
              u0_ref, u2_ref, un0_ref, un1_ref, norm_ref):
  a = jnp.concatenate([a0_ref[...] + y0_ref[...],
                       a1_ref[...] + y1_ref[...]], axis=1)
  degb = deg_ref[...]
  h = a / (degb + 1.0) + b_ref[...]
  h = jnp.where(h > 0, h, 0.01 * h)
  p = jnp.dot(h, wc_ref[...], precision=_HI)
  norm = lax.rsqrt(jnp.maximum(degb, 1.0))
  u0_ref[...] = p[:, 0:32]
  u2_ref[...] = p[:, 64:96]
  un0_ref[...] = p[:, 32:64] * norm
  un1_ref[...] = p[:, 64:96] * norm
  norm_ref[...] = norm


def _tc3_body(s2a_ref, norm_ref, v0_ref, v1_ref):
  norm = norm_ref[...]
  v = -s2a_ref[...] * (norm * norm)
  v0_ref[...] = v[:, 0:16]
  v1_ref[...] = v[:, 16:32]


def _tc4_body(u0_ref, u2_ref, s1_ref, sb0_ref, sb1_ref, norm_ref,
              bc_ref, wf_ref, bf_ref, out_ref):
  norm = norm_ref[...]
  s2b = jnp.concatenate([sb0_ref[...], sb1_ref[...]], axis=1)
  h2 = (u0_ref[...] - (s1_ref[...] + 2.0 * s2b) * norm - u2_ref[...]
        + bc_ref[...])
  h2 = jnp.where(h2 > 0, h2, 0.01 * h2)
  out_ref[...] = jnp.dot(h2, wf_ref[...], precision=_HI) + bf_ref[...]


def _row_spec(cols):
  return pl.BlockSpec((_RB, cols), lambda i: (i, 0))


def _full_spec(shape):
  return pl.BlockSpec(shape, lambda i: tuple(0 for _ in shape))


_tc1 = pl.pallas_call(
    _tc1_body,
    grid=(_GRID,),
    in_specs=[_row_spec(256), _full_spec((256, 128)), _full_spec((256, 128))],
    out_specs=[_row_spec(128), _row_spec(128)],
    out_shape=[jax.ShapeDtypeStruct((N_NODES, 128), F32)] * 2,
)

_tc2 = pl.pallas_call(
    _tc2_body,
    grid=(_GRID,),
    in_specs=[_row_spec(128), _row_spec(128), _row_spec(128), _row_spec(128),
              _row_spec(1), _full_spec((1, 256)), _full_spec((256, 96))],
    out_specs=[_row_spec(32), _row_spec(32), _row_spec(32), _row_spec(32),
               _row_spec(1)],
    out_shape=[jax.ShapeDtypeStruct((N_NODES, 32), F32)] * 4
    + [jax.ShapeDtypeStruct((N_NODES, 1), F32)],
)

_tc3 = pl.pallas_call(
    _tc3_body,
    grid=(_GRID,),
    in_specs=[_row_spec(32), _row_spec(1)],
    out_specs=[_row_spec(16), _row_spec(16)],
    out_shape=[jax.ShapeDtypeStruct((N_NODES, 16), F32)] * 2,
)

_tc4 = pl.pallas_call(
    _tc4_body,
    grid=(_GRID,),
    in_specs=[_row_spec(32), _row_spec(32), _row_spec(32), _row_spec(16),
              _row_spec(16), _row_spec(1), _full_spec((1, 32)),
              _full_spec((32, 40)), _full_spec((1, 40))],
    out_specs=_row_spec(40),
    out_shape=jax.ShapeDtypeStruct((N_NODES, 40), F32),
)


def kernel(features, edge_index, W_sage, b_sage, W_cheb, b_cheb, W_fc, b_fc):
  ei = edge_index.astype(jnp.int32)
  src = ei[0].reshape(NS, NCH, CHUNK)
  dst = ei[1].reshape(NS, NCH, CHUNK)

  y0, y1 = _tc1(features, W_sage[:, :128], W_sage[:, 128:])
  agg0, agg1, deg = _sc_sage(y0, y1, src, dst)

  wc96 = jnp.concatenate([W_cheb[0:256], W_cheb[256:512], W_cheb[512:768]],
                         axis=1)
  u0, u2, un1, un2, norm = _tc2(agg0, agg1, y0, y1, deg.reshape(N_NODES, 1),
                                b_sage.reshape(1, 256), wc96)
  s1, s2a = _sc_cheb1(un1, un2, src, dst)
  v0, v1 = _tc3(s2a, norm)
  s2b0, s2b1 = _sc_cheb2(v0, v1, src, dst)
  out = _tc4(u0, u2, s1, s2b0, s2b1, norm, b_cheb.reshape(1, 32), W_fc,
             b_fc.reshape(1, 40))
  return out
